# baseline (speedup 1.0000x reference)
import dataclasses
import functools

import jax
import jax.numpy as jnp
from jax import lax
from jax.experimental import pallas as pl
from jax.experimental.pallas import tpu as pltpu
from jax.experimental.pallas import tpu_sc as plsc

B = 16384
NUM_CORES = 2
NUM_SUBCORES = 16
LANES = 16
NUM_WORKERS = NUM_CORES * NUM_SUBCORES
CHUNK = B // NUM_WORKERS

_COMPILER_PARAMS = pltpu.CompilerParams()
if "needs_layout_passes" in pltpu.CompilerParams.__dataclass_fields__:
    _COMPILER_PARAMS = dataclasses.replace(
        _COMPILER_PARAMS, needs_layout_passes=False)

_MESH = plsc.VectorSubcoreMesh(
    core_axis_name="c", subcore_axis_name="s",
    num_cores=NUM_CORES, num_subcores=NUM_SUBCORES,
)


@functools.partial(
    pl.kernel,
    out_type=jax.ShapeDtypeStruct((B,), jnp.float32),
    mesh=_MESH,
    scratch_types=[
        pltpu.VMEM((CHUNK,), jnp.int32),
        pltpu.VMEM((CHUNK,), jnp.float32),
        pltpu.VMEM((LANES,), jnp.float32),
    ],
    compiler_params=_COMPILER_PARAMS,
)
def _sc_gather(cs_hbm, ptab_hbm, out_hbm, idx_v, out_v, p_v):
    wid = lax.axis_index("s") * NUM_CORES + lax.axis_index("c")
    base = wid * CHUNK
    pltpu.sync_copy(ptab_hbm, p_v)
    pltpu.sync_copy(cs_hbm.at[pl.ds(base, CHUNK)], idx_v)

    @pl.loop(0, CHUNK, step=LANES)
    def _(i):
        out_v[pl.ds(i, LANES)] = plsc.load_gather(p_v, [idx_v[pl.ds(i, LANES)]])

    pltpu.sync_copy(out_v, out_hbm.at[pl.ds(base, CHUNK)])


@jax.jit
def kernel(cs, xs, params):
    del xs
    cs32 = cs.astype(jnp.int32)
    ptab = jnp.pad(jnp.reshape(params.astype(jnp.float32), (-1,)),
                   (0, LANES - params.size))
    return _sc_gather(cs32, ptab)


# speedup vs baseline: 1.0369x; 1.0369x over previous
import functools

import jax
import jax.numpy as jnp
from jax import lax
from jax.experimental import pallas as pl
from jax.experimental.pallas import tpu as pltpu
from jax.experimental.pallas import tpu_sc as plsc

B = 16384
NUM_CORES = 2
NUM_SUBCORES = 16
LANES = 16
NUM_WORKERS = NUM_CORES * NUM_SUBCORES
CHUNK = B // NUM_WORKERS

_MESH = plsc.VectorSubcoreMesh(
    core_axis_name="c", subcore_axis_name="s",
    num_cores=NUM_CORES, num_subcores=NUM_SUBCORES,
)


@functools.partial(
    pl.kernel,
    out_type=jax.ShapeDtypeStruct((B,), jnp.float32),
    mesh=_MESH,
    scratch_types=[
        pltpu.VMEM((CHUNK,), jnp.int32),
        pltpu.VMEM((CHUNK,), jnp.float32),
        pltpu.VMEM((2 * LANES,), jnp.float32),
        pltpu.SemaphoreType.DMA,
        pltpu.SemaphoreType.DMA,
    ],
)
def _sc_gather(cs_hbm, p_hbm, out_hbm, idx_v, out_v, p_v, sem_i, sem_p):
    wid = lax.axis_index("s") * NUM_CORES + lax.axis_index("c")
    base = wid * CHUNK
    cp_i = pltpu.async_copy(cs_hbm.at[pl.ds(base, CHUNK)], idx_v, sem_i)
    cp_p = pltpu.async_copy(p_hbm, p_v, sem_p)
    cp_p.wait()
    cp_i.wait()
    pv0 = p_v[pl.ds(0, LANES)]
    pv1 = p_v[pl.ds(LANES, LANES)]

    @pl.loop(0, CHUNK, step=LANES)
    def _(i):
        iv = idx_v[pl.ds(i, LANES)]
        out_v[pl.ds(i, LANES)] = jnp.where(iv == 0, pv0, pv1)

    pltpu.sync_copy(out_v, out_hbm.at[pl.ds(base, CHUNK)])


@jax.jit
def kernel(cs, xs, params):
    del xs
    ptab = jnp.reshape(jnp.broadcast_to(params, (2, LANES)), (-1,))
    return _sc_gather(cs.astype(jnp.int32), ptab)


# speedup vs baseline: 1.1129x; 1.0733x over previous
import functools

import jax
import jax.numpy as jnp
from jax import lax
from jax.experimental import pallas as pl
from jax.experimental.pallas import tpu as pltpu
from jax.experimental.pallas import tpu_sc as plsc

B = 16384
NUM_CORES = 1
NUM_SUBCORES = 16
LANES = 16
NUM_WORKERS = NUM_CORES * NUM_SUBCORES
CHUNK = B // NUM_WORKERS

_MESH = plsc.VectorSubcoreMesh(
    core_axis_name="c", subcore_axis_name="s",
    num_cores=NUM_CORES, num_subcores=NUM_SUBCORES,
)


@functools.partial(
    pl.kernel,
    out_type=jax.ShapeDtypeStruct((B,), jnp.float32),
    mesh=_MESH,
    scratch_types=[
        pltpu.VMEM((CHUNK,), jnp.int32),
        pltpu.VMEM((CHUNK,), jnp.float32),
        pltpu.VMEM((2 * LANES,), jnp.float32),
        pltpu.SemaphoreType.DMA,
        pltpu.SemaphoreType.DMA,
    ],
)
def _sc_gather(cs_hbm, p_hbm, out_hbm, idx_v, out_v, p_v, sem_i, sem_p):
    wid = lax.axis_index("s") * NUM_CORES + lax.axis_index("c")
    base = wid * CHUNK
    cp_i = pltpu.async_copy(cs_hbm.at[pl.ds(base, CHUNK)], idx_v, sem_i)
    cp_p = pltpu.async_copy(p_hbm, p_v, sem_p)
    cp_p.wait()
    cp_i.wait()
    pv0 = p_v[pl.ds(0, LANES)]
    pv1 = p_v[pl.ds(LANES, LANES)]

    @pl.loop(0, CHUNK, step=LANES)
    def _(i):
        iv = idx_v[pl.ds(i, LANES)]
        out_v[pl.ds(i, LANES)] = jnp.where(iv == 0, pv0, pv1)

    pltpu.sync_copy(out_v, out_hbm.at[pl.ds(base, CHUNK)])


@jax.jit
def kernel(cs, xs, params):
    del xs
    ptab = jnp.reshape(jnp.broadcast_to(params, (2, LANES)), (-1,))
    return _sc_gather(cs.astype(jnp.int32), ptab)
